# Initial kernel scaffold; baseline (speedup 1.0000x reference)
import jax
import jax.numpy as jnp
from jax.experimental import pallas as pl


def kernel(batch, W_ih, W_hh, b):
    raise NotImplementedError("write your pallas kernel here")


# speedup vs baseline: 3.7626x; 3.7626x over previous
import jax
import jax.numpy as jnp
from jax.experimental import pallas as pl
from jax.experimental.pallas import tpu as pltpu

B = 16
T = 512
I = 512
H = 512
T_CHUNK = 64


def _inproj_kernel(x_ref, wih_ref, bias_ref, out_ref):
    x = x_ref[0]
    y = jax.lax.dot_general(
        x, wih_ref[...], (((1,), (1,)), ((), ())),
        preferred_element_type=jnp.float32,
    )
    out_ref[...] = y + bias_ref[...]


def _rnn_step_kernel(xw_ref, whh_ref, out_ref, h_ref):
    t = pl.program_id(0)

    @pl.when(t == 0)
    def _():
        h_ref[...] = jnp.zeros_like(h_ref)

    h = h_ref[...]
    acc = jax.lax.dot_general(
        h, whh_ref[...], (((1,), (1,)), ((), ())),
        preferred_element_type=jnp.float32,
    )
    h_new = jnp.tanh(acc + xw_ref[0])
    h_ref[...] = h_new
    out_ref[...] = h_new


def kernel(batch, W_ih, W_hh, b):
    bias2d = b.reshape(1, H)

    xw2d = pl.pallas_call(
        _inproj_kernel,
        grid=(B, T // T_CHUNK),
        in_specs=[
            pl.BlockSpec((1, T_CHUNK, I), lambda bi, ti: (bi, ti, 0)),
            pl.BlockSpec((H, I), lambda bi, ti: (0, 0)),
            pl.BlockSpec((1, H), lambda bi, ti: (0, 0)),
        ],
        out_specs=pl.BlockSpec((T_CHUNK, H), lambda bi, ti: (ti, bi)),
        out_shape=jax.ShapeDtypeStruct((T, B * H), jnp.float32),
    )(batch, W_ih, bias2d)

    xw = xw2d.reshape(T, B, H)

    out2d = pl.pallas_call(
        _rnn_step_kernel,
        grid=(T,),
        in_specs=[
            pl.BlockSpec((1, B, H), lambda t: (t, 0, 0)),
            pl.BlockSpec((H, H), lambda t: (0, 0)),
        ],
        out_specs=pl.BlockSpec((B, H), lambda t: (0, t)),
        out_shape=jax.ShapeDtypeStruct((B, T * H), jnp.float32),
        scratch_shapes=[pltpu.VMEM((B, H), jnp.float32)],
    )(xw, W_hh)

    outputs = out2d.reshape(B, T, H)
    hT = outputs[:, -1, :]
    return outputs, hT


# speedup vs baseline: 10.9427x; 2.9082x over previous
import jax
import jax.numpy as jnp
from jax.experimental import pallas as pl
from jax.experimental.pallas import tpu as pltpu

B = 16
T = 512
I = 512
H = 512
T_CHUNK = 64


def _rnn_chunk_kernel(x_ref, wih_ref, whh_ref, bias_ref, out_ref, h_ref, xw_ref):
    @pl.when(pl.program_id(0) == 0)
    def _():
        h_ref[...] = jnp.zeros_like(h_ref)

    x = x_ref[...].reshape(B * T_CHUNK, I)
    xw = jax.lax.dot_general(
        x, wih_ref[...], (((1,), (1,)), ((), ())),
        preferred_element_type=jnp.float32,
    )
    xw_ref[...] = xw.reshape(B, T_CHUNK, H) + bias_ref[...].reshape(1, 1, H)

    whh = whh_ref[...]

    def body(i, h):
        acc = jax.lax.dot_general(
            h, whh, (((1,), (1,)), ((), ())),
            preferred_element_type=jnp.float32,
        )
        h_new = jnp.tanh(acc + xw_ref[:, i, :])
        out_ref[:, pl.ds(i * H, H)] = h_new
        return h_new

    h_ref[...] = jax.lax.fori_loop(0, T_CHUNK, body, h_ref[...], unroll=4)


def kernel(batch, W_ih, W_hh, b):
    bias2d = b.reshape(1, H)

    out2d = pl.pallas_call(
        _rnn_chunk_kernel,
        grid=(T // T_CHUNK,),
        in_specs=[
            pl.BlockSpec((B, T_CHUNK, I), lambda c: (0, c, 0)),
            pl.BlockSpec((H, I), lambda c: (0, 0)),
            pl.BlockSpec((H, H), lambda c: (0, 0)),
            pl.BlockSpec((1, H), lambda c: (0, 0)),
        ],
        out_specs=pl.BlockSpec((B, T_CHUNK * H), lambda c: (0, c)),
        out_shape=jax.ShapeDtypeStruct((B, T * H), jnp.float32),
        scratch_shapes=[
            pltpu.VMEM((B, H), jnp.float32),
            pltpu.VMEM((B, T_CHUNK, H), jnp.float32),
        ],
    )(batch, W_ih, W_hh, bias2d)

    outputs = out2d.reshape(B, T, H)
    hT = outputs[:, -1, :]
    return outputs, hT
